# baseline (speedup 1.0000x reference)
import jax
import jax.numpy as jnp
from jax import lax
from jax.experimental import pallas as pl
from jax.experimental.pallas import tpu as pltpu
from jax.experimental.pallas import tpu_sc as plsc

N, P2, W2, CKV, TOPK = 8, 49, 64, 384, 4
R = N * P2 * TOPK
D = W2 * CKV
NC, NS, L = 2, 16, 16
NW = NC * NS
RPW = R // NW
IST = 8


def _body(gidx_hbm, wrow_hbm, kv_hbm, out_hbm,
          gidx_v, wrow_v, buf, gsem, osem):
    wid = lax.axis_index("s") * NC + lax.axis_index("c")
    base = wid * RPW

    pltpu.sync_copy(gidx_hbm.at[wid], gidx_v)
    pltpu.sync_copy(wrow_hbm.at[wid], wrow_v)

    def row(j, carry):
        gi = gidx_v.at[pl.ds(j * IST, 1)]
        pltpu.async_copy(kv_hbm.at[gi], buf, gsem).wait()
        wb = wrow_v[pl.ds(j * L, L)]

        def mul(q, c):
            buf[0, pl.ds(q * L, L)] = buf[0, pl.ds(q * L, L)] * wb
            return c
        lax.fori_loop(0, D // L, mul, 0)
        pltpu.async_copy(buf, out_hbm.at[pl.ds(base + j, 1)], osem).wait()
        return carry

    lax.fori_loop(0, RPW, row, 0)


@jax.jit
def _sc_gather(gidx, wrow, kvf):
    mesh = plsc.VectorSubcoreMesh(
        core_axis_name="c", subcore_axis_name="s",
        num_cores=NC, num_subcores=NS)
    return pl.kernel(
        _body,
        out_type=jax.ShapeDtypeStruct((R, D), jnp.float32),
        mesh=mesh,
        scratch_types=[
            pltpu.VMEM((RPW * IST,), jnp.int32),
            pltpu.VMEM((RPW * L,), jnp.float32),
            pltpu.VMEM((1, D), jnp.float32),
            pltpu.SemaphoreType.DMA,
            pltpu.SemaphoreType.DMA,
        ],
    )(gidx, wrow, kvf)


def kernel(r_idx, r_weight, kv):
    flat = (r_idx.astype(jnp.int32)
            + jnp.arange(N, dtype=jnp.int32)[:, None, None] * P2)
    gidx = jnp.zeros((NW, RPW, IST), jnp.int32)
    gidx = gidx.at[:, :, 0].set(flat.reshape(NW, RPW)).reshape(NW, RPW * IST)
    wrow = jnp.broadcast_to(
        r_weight.astype(jnp.float32).reshape(NW, RPW, 1),
        (NW, RPW, L)).reshape(NW, RPW * L)
    kvf = kv.reshape(N * P2, D)
    out = _sc_gather(gidx, wrow, kvf)
    return out.reshape(N, P2, TOPK, W2, CKV)


# speedup vs baseline: 2.3568x; 2.3568x over previous
import jax
import jax.numpy as jnp
from jax import lax
from jax.experimental import pallas as pl
from jax.experimental.pallas import tpu as pltpu
from jax.experimental.pallas import tpu_sc as plsc

N, P2, W2, CKV, TOPK = 8, 49, 64, 384, 4
R = N * P2 * TOPK
D = W2 * CKV
NC, NS, L = 2, 16, 16
NW = NC * NS
RPW = R // NW
IST = 8
NBUF = 4
UNROLL = 8


def _body(gidx_hbm, wrow_hbm, kv_hbm, out_hbm,
          gidx_v, wrow_v, bufs, gsems, osems):
    wid = lax.axis_index("s") * NC + lax.axis_index("c")
    base = wid * RPW

    pltpu.sync_copy(gidx_hbm.at[wid], gidx_v)
    pltpu.sync_copy(wrow_hbm.at[wid], wrow_v)

    def start_gather(j, b):
        gi = gidx_v.at[pl.ds(j * IST, 1)]
        pltpu.async_copy(kv_hbm.at[gi], bufs[b], gsems[b])

    def wait_gather(b):
        pltpu.make_async_copy(kv_hbm.at[pl.ds(0, 1)], bufs[b], gsems[b]).wait()

    def start_out(j, b):
        pltpu.async_copy(bufs[b], out_hbm.at[pl.ds(base + j, 1)], osems[b])

    def wait_out(b):
        pltpu.make_async_copy(bufs[b], out_hbm.at[pl.ds(0, 1)], osems[b]).wait()

    def scale(j, b):
        wb = wrow_v[pl.ds(j * L, L)]
        buf = bufs[b]

        def mul(q, c):
            o = q * (L * UNROLL)
            for u in range(UNROLL):
                buf[0, pl.ds(o + u * L, L)] = buf[0, pl.ds(o + u * L, L)] * wb
            return c
        lax.fori_loop(0, D // (L * UNROLL), mul, 0)

    for b in range(NBUF):
        start_gather(b, b)

    def step(i, carry):
        for b in range(NBUF):
            j = i * NBUF + b

            @pl.when(j < RPW)
            def _():
                wait_gather(b)
                scale(j, b)
                start_out(j, b)

            @pl.when(j + NBUF < RPW)
            def _():
                wait_out(b)
                start_gather(j + NBUF, b)
        return carry

    nsteps = (RPW + NBUF - 1) // NBUF
    lax.fori_loop(0, nsteps, step, 0)

    for b in range(NBUF):
        wait_out(b)


@jax.jit
def _sc_gather(gidx, wrow, kvf):
    mesh = plsc.VectorSubcoreMesh(
        core_axis_name="c", subcore_axis_name="s",
        num_cores=NC, num_subcores=NS)
    return pl.kernel(
        _body,
        out_type=jax.ShapeDtypeStruct((R, D), jnp.float32),
        mesh=mesh,
        scratch_types=[
            pltpu.VMEM((RPW * IST,), jnp.int32),
            pltpu.VMEM((RPW * L,), jnp.float32),
            [pltpu.VMEM((1, D), jnp.float32) for _ in range(NBUF)],
            [pltpu.SemaphoreType.DMA for _ in range(NBUF)],
            [pltpu.SemaphoreType.DMA for _ in range(NBUF)],
        ],
    )(gidx, wrow, kvf)


def kernel(r_idx, r_weight, kv):
    flat = (r_idx.astype(jnp.int32)
            + jnp.arange(N, dtype=jnp.int32)[:, None, None] * P2)
    gidx = jnp.zeros((NW, RPW, IST), jnp.int32)
    gidx = gidx.at[:, :, 0].set(flat.reshape(NW, RPW)).reshape(NW, RPW * IST)
    wrow = jnp.broadcast_to(
        r_weight.astype(jnp.float32).reshape(NW, RPW, 1),
        (NW, RPW, L)).reshape(NW, RPW * L)
    kvf = kv.reshape(N * P2, D)
    out = _sc_gather(gidx, wrow, kvf)
    return out.reshape(N, P2, TOPK, W2, CKV)


# speedup vs baseline: 5.3437x; 2.2674x over previous
import jax
import jax.numpy as jnp
from jax import lax
from jax.experimental import pallas as pl
from jax.experimental.pallas import tpu as pltpu
from jax.experimental.pallas import tpu_sc as plsc

N, P2, W2, CKV, TOPK = 8, 49, 64, 384, 4
R = N * P2 * TOPK
D = W2 * CKV
NC, NS, L = 2, 16, 16
NW = NC * NS
RPW = R // NW
NBUF = 4
CPR = CKV // L


def _body(gidx_hbm, wrow_hbm, kv_hbm, out_hbm,
          gidx_v, wrow_v, bufs, gsems, osems):
    wid = lax.axis_index("s") * NC + lax.axis_index("c")
    base = wid * RPW

    pltpu.sync_copy(gidx_hbm.at[wid], gidx_v)
    pltpu.sync_copy(wrow_hbm.at[wid], wrow_v)

    def start_gather(j, b):
        gi = gidx_v.at[(j * 8) // 128, pl.ds((j * 8) % 128, 1)]
        pltpu.async_copy(kv_hbm.at[gi], bufs[b], gsems[b])

    def wait_gather(b):
        pltpu.make_async_copy(kv_hbm.at[pl.ds(0, 1)], bufs[b], gsems[b]).wait()

    def start_out(j, b):
        pltpu.async_copy(bufs[b], out_hbm.at[pl.ds(base + j, 1)], osems[b])

    def wait_out(b):
        pltpu.make_async_copy(bufs[b], out_hbm.at[pl.ds(0, 1)], osems[b]).wait()

    def scale(j, b):
        wb = wrow_v[(j * 16) // 128, pl.ds((j * 16) % 128, 16)]
        buf = bufs[b]

        def mul(r, c):
            for u in range(CPR):
                buf[0, r, pl.ds(u * L, L)] = buf[0, r, pl.ds(u * L, L)] * wb
            return c
        lax.fori_loop(0, W2, mul, 0)

    start_gather(0, 0)
    start_gather(1, 1)
    for j in range(RPW):
        nxt = j + 2
        if nxt < RPW:
            b2 = nxt % NBUF
            if nxt >= NBUF:
                wait_out(b2)
            start_gather(nxt, b2)
        b = j % NBUF
        wait_gather(b)
        scale(j, b)
        start_out(j, b)

    for b in range(NBUF):
        wait_out(b)


@jax.jit
def _sc_gather(gidx, wrow, kvf):
    mesh = plsc.VectorSubcoreMesh(
        core_axis_name="c", subcore_axis_name="s",
        num_cores=NC, num_subcores=NS)
    return pl.kernel(
        _body,
        out_type=jax.ShapeDtypeStruct((R, W2, CKV), jnp.float32),
        mesh=mesh,
        compiler_params=pltpu.CompilerParams(use_tc_tiling_on_sc=True),
        scratch_types=[
            pltpu.VMEM((8, 128), jnp.int32),
            pltpu.VMEM((8, 128), jnp.float32),
            [pltpu.VMEM((1, W2, CKV), jnp.float32) for _ in range(NBUF)],
            [pltpu.SemaphoreType.DMA for _ in range(NBUF)],
            [pltpu.SemaphoreType.DMA for _ in range(NBUF)],
        ],
    )(gidx, wrow, kvf)


def kernel(r_idx, r_weight, kv):
    flat = (r_idx.astype(jnp.int32)
            + jnp.arange(N, dtype=jnp.int32)[:, None, None] * P2)
    gidx = jnp.zeros((NW, 8 * 128), jnp.int32)
    gidx = gidx.at[:, jnp.arange(RPW) * 8].set(flat.reshape(NW, RPW))
    gidx = gidx.reshape(NW, 8, 128)
    wrow = jnp.pad(
        jnp.broadcast_to(
            r_weight.astype(jnp.float32).reshape(NW, RPW, 1),
            (NW, RPW, L)).reshape(NW, RPW * L),
        ((0, 0), (0, 8 * 128 - RPW * L))).reshape(NW, 8, 128)
    kvf = kv.reshape(N * P2, W2, CKV)
    out = _sc_gather(gidx, wrow, kvf)
    return out.reshape(N, P2, TOPK, W2, CKV)
